# baseline (speedup 1.0000x reference)
import functools

import jax
import jax.numpy as jnp
from jax import lax
from jax.experimental import pallas as pl
from jax.experimental.pallas import tpu as pltpu
from jax.experimental.pallas import tpu_sc as plsc

EPS = 1e-6
NC = 2
NS = 16
K = 128


def _bcos_block(xb, w):
    lin = lax.dot_general(xb, w, (((1,), (1,)), ((), ())),
                          preferred_element_type=jnp.float32)
    nx2 = jnp.square(jnp.maximum(
        jnp.sqrt(jnp.sum(xb * xb, axis=1, keepdims=True)), 1e-12))
    nw2 = jnp.square(jnp.maximum(
        jnp.sqrt(jnp.sum(w * w, axis=1, keepdims=True)), 1e-12))
    denom2 = lax.dot_general(nx2, nw2, (((1,), (1,)), ((), ())),
                             preferred_element_type=jnp.float32)
    cos = jnp.maximum(lin / jnp.sqrt(denom2), EPS)
    return lin * cos


def _tc1_body(x_ref, wn_ref, ws_ref, m_ref, s_ref):
    xb = x_ref[...]
    m_ref[...] = _bcos_block(xb, wn_ref[...])
    s_ref[...] = lax.dot_general(xb, ws_ref[...], (((1,), (1,)), ((), ())),
                                 preferred_element_type=jnp.float32)


def _tc2_body(p0_ref, p1_ref, d0_ref, d1_ref, s1_ref, wn_ref, ws_ref,
              m_ref, s_ref):
    deg = jnp.maximum(d0_ref[:, 0:1] + d1_ref[:, 0:1], 1.0)
    agg = (p0_ref[...] + p1_ref[...]) / deg
    h = jnp.maximum(agg + s1_ref[...], 0.0)
    m_ref[...] = _bcos_block(h, wn_ref[...])
    s_ref[...] = lax.dot_general(h, ws_ref[...], (((1,), (1,)), ((), ())),
                                 preferred_element_type=jnp.float32)


def _tc3_body(p0_ref, p1_ref, d0_ref, d1_ref, s2_ref, o_ref):
    deg = jnp.maximum(d0_ref[:, 0:1] + d1_ref[:, 0:1], 1.0)
    o_ref[...] = (p0_ref[...] + p1_ref[...]) / deg + s2_ref[...]


def _row_block(n):
    for b in (2048, 2000, 1024, 1000, 512, 500, 400, 256, 200, 128, 80, 64,
              40, 32, 16, 8):
        if n % b == 0:
            return b
    return n


def _sc_mesh():
    return plsc.VectorSubcoreMesh(core_axis_name="c", subcore_axis_name="s")


def _make_sc_scatter(npad, cpw):
    gb = 8
    assert cpw % gb == 0
    rps = npad // NS
    nb = rps // K

    @functools.partial(
        pl.kernel, mesh=_sc_mesh(),
        out_type=[jax.ShapeDtypeStruct((NC, npad, 128), jnp.float32)],
        scratch_types=[
            pltpu.VMEM((gb, K), jnp.int32),
            pltpu.VMEM((gb, K), jnp.int32),
            pltpu.VMEM((K, 128), jnp.float32),
            pltpu.VMEM_SHARED((npad, 128), jnp.float32),
            pltpu.SemaphoreType.DMA,
        ])
    def sc_fn(m_hbm, row_hbm, col_hbm, z_hbm, out_hbm,
              colv, rowv, rows, acc, sem):
        cid = lax.axis_index("c")
        sid = lax.axis_index("s")
        wid = sid * NC + cid

        pltpu.sync_copy(z_hbm, rows)

        def zb(i, c):
            off = pl.multiple_of(sid * rps + i * K, K)
            pltpu.sync_copy(rows, acc.at[pl.ds(off, K)])
            return c
        lax.fori_loop(0, nb, zb, 0)
        plsc.subcore_barrier()

        base = wid * cpw

        def group(gidx, c):
            cb = pl.multiple_of(base + gidx * gb, gb)
            pltpu.sync_copy(col_hbm.at[pl.ds(cb, gb)], colv)
            pltpu.sync_copy(row_hbm.at[pl.ds(cb, gb)], rowv)
            for q in range(gb):
                pltpu.async_copy(m_hbm.at[colv.at[q]], rows, sem).wait()
                pltpu.sync_copy(rows, acc.at[rowv.at[q]], add=True)
            return c
        lax.fori_loop(0, cpw // gb, group, 0)
        plsc.subcore_barrier()

        def wb(i, c):
            off = pl.multiple_of(sid * rps + i * K, K)
            pltpu.sync_copy(acc.at[pl.ds(off, K)], rows)
            pltpu.sync_copy(rows, out_hbm.at[cid, pl.ds(off, K)])
            return c
        lax.fori_loop(0, nb, wb, 0)

    return sc_fn


def _make_sc_deg(npad, cpw):
    rps = npad // NS
    nb = rps // K

    gb = 8
    assert cpw % gb == 0

    @functools.partial(
        pl.kernel, mesh=_sc_mesh(),
        out_type=[jax.ShapeDtypeStruct((NC, npad, 128), jnp.float32)],
        scratch_types=[
            pltpu.VMEM((gb, K), jnp.int32),
            pltpu.VMEM((K, 128), jnp.float32),
            pltpu.VMEM_SHARED((npad, 128), jnp.float32),
        ])
    def sc_fn(row_hbm, z_hbm, ones_hbm, out_hbm, rowv, rows, acc):
        cid = lax.axis_index("c")
        sid = lax.axis_index("s")
        wid = sid * NC + cid

        pltpu.sync_copy(z_hbm, rows)

        def zb(i, c):
            off = pl.multiple_of(sid * rps + i * K, K)
            pltpu.sync_copy(rows, acc.at[pl.ds(off, K)])
            return c
        lax.fori_loop(0, nb, zb, 0)
        pltpu.sync_copy(ones_hbm, rows)
        plsc.subcore_barrier()

        base = wid * cpw

        def group(gidx, c):
            cb = pl.multiple_of(base + gidx * gb, gb)
            pltpu.sync_copy(row_hbm.at[pl.ds(cb, gb)], rowv)
            for q in range(gb):
                pltpu.sync_copy(rows, acc.at[rowv.at[q]], add=True)
            return c
        lax.fori_loop(0, cpw // gb, group, 0)
        plsc.subcore_barrier()

        def wb(i, c):
            off = pl.multiple_of(sid * rps + i * K, K)
            pltpu.sync_copy(acc.at[pl.ds(off, K)], rows)
            pltpu.sync_copy(rows, out_hbm.at[cid, pl.ds(off, K)])
            return c
        lax.fori_loop(0, nb, wb, 0)

    return sc_fn


def kernel(x, edge_index, Wn1, Wn2, Ws1, Ws2):
    n, d = x.shape
    e = edge_index.shape[1]
    nwork = NC * NS
    cpw = -(-e // (nwork * K))
    cpw += cpw % 2
    epad = nwork * cpw * K
    npad = -(-(n + 1) // (NS * K)) * NS * K

    row = edge_index[0]
    col = edge_index[1]
    pad = epad - e
    padrow = n + jnp.arange(pad, dtype=jnp.int32) % (npad - n)
    rowp = jnp.concatenate([row, padrow])
    colp = jnp.concatenate([col, jnp.zeros((pad,), jnp.int32)])
    row2 = rowp.reshape(-1, K)
    col2 = colp.reshape(-1, K)
    z128 = jnp.zeros((K, 128), jnp.float32)
    o128 = jnp.ones((K, 128), jnp.float32)

    bm = _row_block(n)
    grid = (n // bm,)
    full_w = pl.BlockSpec((d, d), lambda i: (0, 0))
    rows128 = pl.BlockSpec((bm, 128), lambda i: (i, 0))

    (degp,) = _make_sc_deg(npad, cpw)(row2, z128, o128)

    m1, s1 = pl.pallas_call(
        _tc1_body,
        grid=grid,
        in_specs=[rows128, full_w, full_w],
        out_specs=[rows128, rows128],
        out_shape=[jax.ShapeDtypeStruct((n, 128), jnp.float32)] * 2,
    )(x, Wn1, Ws1)

    sc_scatter = _make_sc_scatter(npad, cpw)
    (aggp,) = sc_scatter(m1, row2, col2, z128)

    m2, s2 = pl.pallas_call(
        _tc2_body,
        grid=grid,
        in_specs=[rows128, rows128, rows128, rows128, rows128,
                  full_w, full_w],
        out_specs=[rows128, rows128],
        out_shape=[jax.ShapeDtypeStruct((n, 128), jnp.float32)] * 2,
    )(aggp[0, :n], aggp[1, :n], degp[0, :n], degp[1, :n], s1, Wn2, Ws2)

    (agg2p,) = sc_scatter(m2, row2, col2, z128)

    out = pl.pallas_call(
        _tc3_body,
        grid=grid,
        in_specs=[rows128, rows128, rows128, rows128, rows128],
        out_specs=rows128,
        out_shape=jax.ShapeDtypeStruct((n, 128), jnp.float32),
    )(agg2p[0, :n], agg2p[1, :n], degp[0, :n], degp[1, :n], s2)

    return out


# speedup vs baseline: 1.3517x; 1.3517x over previous
import functools

import jax
import jax.numpy as jnp
from jax import lax
from jax.experimental import pallas as pl
from jax.experimental.pallas import tpu as pltpu
from jax.experimental.pallas import tpu_sc as plsc

EPS = 1e-6
NC = 2
NS = 16
K = 128


def _bcos_block(xb, w):
    lin = lax.dot_general(xb, w, (((1,), (1,)), ((), ())),
                          preferred_element_type=jnp.float32)
    nx2 = jnp.square(jnp.maximum(
        jnp.sqrt(jnp.sum(xb * xb, axis=1, keepdims=True)), 1e-12))
    nw2 = jnp.square(jnp.maximum(
        jnp.sqrt(jnp.sum(w * w, axis=1, keepdims=True)), 1e-12))
    denom2 = lax.dot_general(nx2, nw2, (((1,), (1,)), ((), ())),
                             preferred_element_type=jnp.float32)
    cos = jnp.maximum(lin / jnp.sqrt(denom2), EPS)
    return lin * cos


def _tc1_body(x_ref, wn_ref, ws_ref, m_ref, s_ref):
    xb = x_ref[...]
    m_ref[...] = _bcos_block(xb, wn_ref[...])
    s_ref[...] = lax.dot_general(xb, ws_ref[...], (((1,), (1,)), ((), ())),
                                 preferred_element_type=jnp.float32)


def _tc2_body(p0_ref, p1_ref, d0_ref, d1_ref, s1_ref, wn_ref, ws_ref,
              m_ref, s_ref):
    deg = jnp.maximum(d0_ref[:, 0:1] + d1_ref[:, 0:1], 1.0)
    agg = (p0_ref[...] + p1_ref[...]) / deg
    h = jnp.maximum(agg + s1_ref[...], 0.0)
    m_ref[...] = _bcos_block(h, wn_ref[...])
    s_ref[...] = lax.dot_general(h, ws_ref[...], (((1,), (1,)), ((), ())),
                                 preferred_element_type=jnp.float32)


def _tc3_body(p0_ref, p1_ref, d0_ref, d1_ref, s2_ref, o_ref):
    deg = jnp.maximum(d0_ref[:, 0:1] + d1_ref[:, 0:1], 1.0)
    o_ref[...] = (p0_ref[...] + p1_ref[...]) / deg + s2_ref[...]


def _row_block(n):
    for b in (2048, 2000, 1024, 1000, 512, 500, 400, 256, 200, 128, 80, 64,
              40, 32, 16, 8):
        if n % b == 0:
            return b
    return n


def _sc_mesh():
    return plsc.VectorSubcoreMesh(core_axis_name="c", subcore_axis_name="s")


def _make_sc_scatter(npad, cpw):
    rps = npad // NS
    nb = rps // K

    @functools.partial(
        pl.kernel, mesh=_sc_mesh(),
        out_type=[jax.ShapeDtypeStruct((NC, npad, 128), jnp.float32)],
        scratch_types=[
            pltpu.VMEM((K,), jnp.int32),
            pltpu.VMEM((K,), jnp.int32),
            pltpu.VMEM((K, 128), jnp.float32),
            pltpu.VMEM_SHARED((npad, 128), jnp.float32),
            pltpu.SemaphoreType.DMA,
            pltpu.SemaphoreType.DMA,
        ])
    def sc_fn(m_hbm, row_hbm, col_hbm, z_hbm, out_hbm,
              colv, rowv, rows, acc, sem, semi):
        cid = lax.axis_index("c")
        sid = lax.axis_index("s")
        wid = sid * NC + cid

        pltpu.sync_copy(z_hbm, rows)

        def zb(i, c):
            off = pl.multiple_of(sid * rps + i * K, K)
            pltpu.sync_copy(rows, acc.at[pl.ds(off, K)])
            return c
        lax.fori_loop(0, nb, zb, 0)
        plsc.subcore_barrier()

        base = wid * cpw

        def chunk(j, c):
            off = pl.multiple_of((base + j) * K, K)
            csrc = col_hbm.at[pl.ds(off, K)]
            rsrc = row_hbm.at[pl.ds(off, K)]
            pltpu.async_copy(csrc, colv, semi)
            pltpu.async_copy(rsrc, rowv, sem)
            pltpu.make_async_copy(csrc, colv, semi).wait()
            pltpu.make_async_copy(rsrc, rowv, sem).wait()
            pltpu.async_copy(m_hbm.at[colv], rows, sem).wait()
            pltpu.sync_copy(rows, acc.at[rowv], add=True)
            return c
        lax.fori_loop(0, cpw, chunk, 0)
        plsc.subcore_barrier()

        def wb(i, c):
            off = pl.multiple_of(sid * rps + i * K, K)
            pltpu.sync_copy(acc.at[pl.ds(off, K)], rows)
            pltpu.sync_copy(rows, out_hbm.at[cid, pl.ds(off, K)])
            return c
        lax.fori_loop(0, nb, wb, 0)

    return sc_fn


def _make_sc_deg(npad, cpw):
    rps = npad // NS
    nb = rps // K

    @functools.partial(
        pl.kernel, mesh=_sc_mesh(),
        out_type=[jax.ShapeDtypeStruct((NC, npad, 128), jnp.float32)],
        scratch_types=[
            pltpu.VMEM((K,), jnp.int32),
            pltpu.VMEM((K, 128), jnp.float32),
            pltpu.VMEM_SHARED((npad, 128), jnp.float32),
        ])
    def sc_fn(row_hbm, z_hbm, ones_hbm, out_hbm, rowv, rows, acc):
        cid = lax.axis_index("c")
        sid = lax.axis_index("s")
        wid = sid * NC + cid

        pltpu.sync_copy(z_hbm, rows)

        def zb(i, c):
            off = pl.multiple_of(sid * rps + i * K, K)
            pltpu.sync_copy(rows, acc.at[pl.ds(off, K)])
            return c
        lax.fori_loop(0, nb, zb, 0)
        pltpu.sync_copy(ones_hbm, rows)
        plsc.subcore_barrier()

        base = wid * cpw

        def chunk(j, c):
            off = pl.multiple_of((base + j) * K, K)
            pltpu.sync_copy(row_hbm.at[pl.ds(off, K)], rowv)
            pltpu.sync_copy(rows, acc.at[rowv], add=True)
            return c
        lax.fori_loop(0, cpw, chunk, 0)
        plsc.subcore_barrier()

        def wb(i, c):
            off = pl.multiple_of(sid * rps + i * K, K)
            pltpu.sync_copy(acc.at[pl.ds(off, K)], rows)
            pltpu.sync_copy(rows, out_hbm.at[cid, pl.ds(off, K)])
            return c
        lax.fori_loop(0, nb, wb, 0)

    return sc_fn


def kernel(x, edge_index, Wn1, Wn2, Ws1, Ws2):
    n, d = x.shape
    e = edge_index.shape[1]
    nwork = NC * NS
    cpw = -(-e // (nwork * K))
    epad = nwork * cpw * K
    npad = -(-(n + 1) // (NS * K)) * NS * K

    row = edge_index[0]
    col = edge_index[1]
    pad = epad - e
    padrow = n + jnp.arange(pad, dtype=jnp.int32) % (npad - n)
    rowp = jnp.concatenate([row, padrow])
    colp = jnp.concatenate([col, jnp.zeros((pad,), jnp.int32)])
    z128 = jnp.zeros((K, 128), jnp.float32)
    o128 = jnp.ones((K, 128), jnp.float32)

    bm = _row_block(n)
    grid = (n // bm,)
    full_w = pl.BlockSpec((d, d), lambda i: (0, 0))
    rows128 = pl.BlockSpec((bm, 128), lambda i: (i, 0))

    (degp,) = _make_sc_deg(npad, cpw)(rowp, z128, o128)

    m1, s1 = pl.pallas_call(
        _tc1_body,
        grid=grid,
        in_specs=[rows128, full_w, full_w],
        out_specs=[rows128, rows128],
        out_shape=[jax.ShapeDtypeStruct((n, 128), jnp.float32)] * 2,
    )(x, Wn1, Ws1)

    sc_scatter = _make_sc_scatter(npad, cpw)
    (aggp,) = sc_scatter(m1, rowp, colp, z128)

    m2, s2 = pl.pallas_call(
        _tc2_body,
        grid=grid,
        in_specs=[rows128, rows128, rows128, rows128, rows128,
                  full_w, full_w],
        out_specs=[rows128, rows128],
        out_shape=[jax.ShapeDtypeStruct((n, 128), jnp.float32)] * 2,
    )(aggp[0, :n], aggp[1, :n], degp[0, :n], degp[1, :n], s1, Wn2, Ws2)

    (agg2p,) = sc_scatter(m2, rowp, colp, z128)

    out = pl.pallas_call(
        _tc3_body,
        grid=grid,
        in_specs=[rows128, rows128, rows128, rows128, rows128],
        out_specs=rows128,
        out_shape=jax.ShapeDtypeStruct((n, 128), jnp.float32),
    )(agg2p[0, :n], agg2p[1, :n], degp[0, :n], degp[1, :n], s2)

    return out


# speedup vs baseline: 1.5339x; 1.1347x over previous
import functools

import jax
import jax.numpy as jnp
from jax import lax
from jax.experimental import pallas as pl
from jax.experimental.pallas import tpu as pltpu
from jax.experimental.pallas import tpu_sc as plsc

EPS = 1e-6
NC = 2
NS = 16
K = 128


def _bcos_block(xb, w):
    lin = lax.dot_general(xb, w, (((1,), (1,)), ((), ())),
                          preferred_element_type=jnp.float32)
    nx2 = jnp.square(jnp.maximum(
        jnp.sqrt(jnp.sum(xb * xb, axis=1, keepdims=True)), 1e-12))
    nw2 = jnp.square(jnp.maximum(
        jnp.sqrt(jnp.sum(w * w, axis=1, keepdims=True)), 1e-12))
    denom2 = lax.dot_general(nx2, nw2, (((1,), (1,)), ((), ())),
                             preferred_element_type=jnp.float32)
    cos = jnp.maximum(lin / jnp.sqrt(denom2), EPS)
    return lin * cos


def _tc1_body(x_ref, wn_ref, ws_ref, m_ref, s_ref):
    xb = x_ref[...]
    m_ref[...] = _bcos_block(xb, wn_ref[...])
    s_ref[...] = lax.dot_general(xb, ws_ref[...], (((1,), (1,)), ((), ())),
                                 preferred_element_type=jnp.float32)


def _tc2_body(p0_ref, p1_ref, d0_ref, d1_ref, s1_ref, wn_ref, ws_ref,
              m_ref, s_ref):
    deg = jnp.maximum(d0_ref[:, 0:1] + d1_ref[:, 0:1], 1.0)
    agg = (p0_ref[...] + p1_ref[...]) / deg
    h = jnp.maximum(agg + s1_ref[...], 0.0)
    m_ref[...] = _bcos_block(h, wn_ref[...])
    s_ref[...] = lax.dot_general(h, ws_ref[...], (((1,), (1,)), ((), ())),
                                 preferred_element_type=jnp.float32)


def _tc3_body(p0_ref, p1_ref, d0_ref, d1_ref, s2_ref, o_ref):
    deg = jnp.maximum(d0_ref[:, 0:1] + d1_ref[:, 0:1], 1.0)
    o_ref[...] = (p0_ref[...] + p1_ref[...]) / deg + s2_ref[...]


def _row_block(n):
    for b in (2048, 2000, 1024, 1000, 512, 500, 400, 256, 200, 128, 80, 64,
              40, 32, 16, 8):
        if n % b == 0:
            return b
    return n


def _sc_mesh():
    return plsc.VectorSubcoreMesh(core_axis_name="c", subcore_axis_name="s")


def _make_sc_scatter(npad, cpw):
    assert cpw % 2 == 1
    rps = npad // NS
    nb = rps // K

    @functools.partial(
        pl.kernel, mesh=_sc_mesh(),
        out_type=[jax.ShapeDtypeStruct((NC, npad, 128), jnp.float32)],
        scratch_types=[
            pltpu.VMEM((K,), jnp.int32),
            pltpu.VMEM((K,), jnp.int32),
            pltpu.VMEM((K,), jnp.int32),
            pltpu.VMEM((K,), jnp.int32),
            pltpu.VMEM((K, 128), jnp.float32),
            pltpu.VMEM_SHARED((npad, 128), jnp.float32),
            pltpu.SemaphoreType.DMA,
            pltpu.SemaphoreType.DMA,
            pltpu.SemaphoreType.DMA,
            pltpu.SemaphoreType.DMA,
            pltpu.SemaphoreType.DMA,
        ])
    def sc_fn(m_hbm, row_hbm, col_hbm, z_hbm, out_hbm,
              colv0, rowv0, colv1, rowv1, rows, acc,
              semg, sc0, sr0, sc1, sr1):
        cid = lax.axis_index("c")
        sid = lax.axis_index("s")
        wid = sid * NC + cid
        bufs = ((colv0, rowv0, sc0, sr0), (colv1, rowv1, sc1, sr1))

        pltpu.sync_copy(z_hbm, rows)

        def zb(i, c):
            off = pl.multiple_of(sid * rps + i * K, K)
            pltpu.sync_copy(rows, acc.at[pl.ds(off, K)])
            return c
        lax.fori_loop(0, nb, zb, 0)
        plsc.subcore_barrier()

        base = wid * cpw

        def start_idx(j, b):
            cv, rv, sc, sr = bufs[b]
            off = pl.multiple_of((base + j) * K, K)
            pltpu.async_copy(col_hbm.at[pl.ds(off, K)], cv, sc)
            pltpu.async_copy(row_hbm.at[pl.ds(off, K)], rv, sr)

        def wait_idx(j, b):
            cv, rv, sc, sr = bufs[b]
            off = pl.multiple_of((base + j) * K, K)
            pltpu.make_async_copy(col_hbm.at[pl.ds(off, K)], cv, sc).wait()
            pltpu.make_async_copy(row_hbm.at[pl.ds(off, K)], rv, sr).wait()

        def work(b):
            cv, rv, _, _ = bufs[b]
            pltpu.async_copy(m_hbm.at[cv], rows, semg).wait()
            pltpu.sync_copy(rows, acc.at[rv], add=True)

        start_idx(0, 0)

        def pair(g, c):
            j = 2 * g
            wait_idx(j, 0)
            start_idx(j + 1, 1)
            work(0)
            wait_idx(j + 1, 1)
            start_idx(j + 2, 0)
            work(1)
            return c
        lax.fori_loop(0, (cpw - 1) // 2, pair, 0)
        wait_idx(cpw - 1, 0)
        work(0)
        plsc.subcore_barrier()

        def wb(i, c):
            off = pl.multiple_of(sid * rps + i * K, K)
            pltpu.sync_copy(acc.at[pl.ds(off, K)], rows)
            pltpu.sync_copy(rows, out_hbm.at[cid, pl.ds(off, K)])
            return c
        lax.fori_loop(0, nb, wb, 0)

    return sc_fn


def _make_sc_deg(npad, cpw):
    rps = npad // NS
    nb = rps // K

    assert cpw % 2 == 1

    @functools.partial(
        pl.kernel, mesh=_sc_mesh(),
        out_type=[jax.ShapeDtypeStruct((NC, npad, 128), jnp.float32)],
        scratch_types=[
            pltpu.VMEM((K,), jnp.int32),
            pltpu.VMEM((K,), jnp.int32),
            pltpu.VMEM((K, 128), jnp.float32),
            pltpu.VMEM_SHARED((npad, 128), jnp.float32),
            pltpu.SemaphoreType.DMA,
            pltpu.SemaphoreType.DMA,
        ])
    def sc_fn(row_hbm, z_hbm, ones_hbm, out_hbm, rowv0, rowv1, rows, acc,
              sr0, sr1):
        cid = lax.axis_index("c")
        sid = lax.axis_index("s")
        wid = sid * NC + cid
        bufs = ((rowv0, sr0), (rowv1, sr1))

        pltpu.sync_copy(z_hbm, rows)

        def zb(i, c):
            off = pl.multiple_of(sid * rps + i * K, K)
            pltpu.sync_copy(rows, acc.at[pl.ds(off, K)])
            return c
        lax.fori_loop(0, nb, zb, 0)
        pltpu.sync_copy(ones_hbm, rows)
        plsc.subcore_barrier()

        base = wid * cpw

        def start_idx(j, b):
            rv, sr = bufs[b]
            off = pl.multiple_of((base + j) * K, K)
            pltpu.async_copy(row_hbm.at[pl.ds(off, K)], rv, sr)

        def wait_idx(j, b):
            rv, sr = bufs[b]
            off = pl.multiple_of((base + j) * K, K)
            pltpu.make_async_copy(row_hbm.at[pl.ds(off, K)], rv, sr).wait()

        def work(b):
            rv, _ = bufs[b]
            pltpu.sync_copy(rows, acc.at[rv], add=True)

        start_idx(0, 0)

        def pair(g, c):
            j = 2 * g
            wait_idx(j, 0)
            start_idx(j + 1, 1)
            work(0)
            wait_idx(j + 1, 1)
            start_idx(j + 2, 0)
            work(1)
            return c
        lax.fori_loop(0, (cpw - 1) // 2, pair, 0)
        wait_idx(cpw - 1, 0)
        work(0)
        plsc.subcore_barrier()

        def wb(i, c):
            off = pl.multiple_of(sid * rps + i * K, K)
            pltpu.sync_copy(acc.at[pl.ds(off, K)], rows)
            pltpu.sync_copy(rows, out_hbm.at[cid, pl.ds(off, K)])
            return c
        lax.fori_loop(0, nb, wb, 0)

    return sc_fn


def kernel(x, edge_index, Wn1, Wn2, Ws1, Ws2):
    n, d = x.shape
    e = edge_index.shape[1]
    nwork = NC * NS
    cpw = -(-e // (nwork * K))
    cpw += 1 - (cpw % 2)
    epad = nwork * cpw * K
    npad = -(-(n + 1) // (NS * K)) * NS * K

    row = edge_index[0]
    col = edge_index[1]
    pad = epad - e
    padrow = n + jnp.arange(pad, dtype=jnp.int32) % (npad - n)
    rowp = jnp.concatenate([row, padrow])
    colp = jnp.concatenate([col, jnp.zeros((pad,), jnp.int32)])
    z128 = jnp.zeros((K, 128), jnp.float32)
    o128 = jnp.ones((K, 128), jnp.float32)

    bm = _row_block(n)
    grid = (n // bm,)
    full_w = pl.BlockSpec((d, d), lambda i: (0, 0))
    rows128 = pl.BlockSpec((bm, 128), lambda i: (i, 0))

    (degp,) = _make_sc_deg(npad, cpw)(rowp, z128, o128)

    m1, s1 = pl.pallas_call(
        _tc1_body,
        grid=grid,
        in_specs=[rows128, full_w, full_w],
        out_specs=[rows128, rows128],
        out_shape=[jax.ShapeDtypeStruct((n, 128), jnp.float32)] * 2,
    )(x, Wn1, Ws1)

    sc_scatter = _make_sc_scatter(npad, cpw)
    (aggp,) = sc_scatter(m1, rowp, colp, z128)

    m2, s2 = pl.pallas_call(
        _tc2_body,
        grid=grid,
        in_specs=[rows128, rows128, rows128, rows128, rows128,
                  full_w, full_w],
        out_specs=[rows128, rows128],
        out_shape=[jax.ShapeDtypeStruct((n, 128), jnp.float32)] * 2,
    )(aggp[0, :n], aggp[1, :n], degp[0, :n], degp[1, :n], s1, Wn2, Ws2)

    (agg2p,) = sc_scatter(m2, rowp, colp, z128)

    out = pl.pallas_call(
        _tc3_body,
        grid=grid,
        in_specs=[rows128, rows128, rows128, rows128, rows128],
        out_specs=rows128,
        out_shape=jax.ShapeDtypeStruct((n, 128), jnp.float32),
    )(agg2p[0, :n], agg2p[1, :n], degp[0, :n], degp[1, :n], s2)

    return out


# speedup vs baseline: 1.7006x; 1.1087x over previous
import functools

import jax
import jax.numpy as jnp
from jax import lax
from jax.experimental import pallas as pl
from jax.experimental.pallas import tpu as pltpu
from jax.experimental.pallas import tpu_sc as plsc

EPS = 1e-6
NC = 2
NS = 16
K = 128


def _bcos_block(xb, w):
    lin = lax.dot_general(xb, w, (((1,), (1,)), ((), ())),
                          preferred_element_type=jnp.float32)
    nx2 = jnp.square(jnp.maximum(
        jnp.sqrt(jnp.sum(xb * xb, axis=1, keepdims=True)), 1e-12))
    nw2 = jnp.square(jnp.maximum(
        jnp.sqrt(jnp.sum(w * w, axis=1, keepdims=True)), 1e-12))
    denom2 = lax.dot_general(nx2, nw2, (((1,), (1,)), ((), ())),
                             preferred_element_type=jnp.float32)
    cos = jnp.maximum(lin / jnp.sqrt(denom2), EPS)
    return lin * cos


def _tc1_body(x_ref, wn_ref, ws_ref, m_ref, s_ref):
    xb = x_ref[...]
    m_ref[...] = _bcos_block(xb, wn_ref[...])
    s_ref[...] = lax.dot_general(xb, ws_ref[...], (((1,), (1,)), ((), ())),
                                 preferred_element_type=jnp.float32)


def _tc2_body(p0_ref, p1_ref, d0_ref, d1_ref, s1_ref, wn_ref, ws_ref,
              m_ref, s_ref):
    deg = jnp.maximum(d0_ref[:, 0:1] + d1_ref[:, 0:1], 1.0)
    agg = (p0_ref[...] + p1_ref[...]) / deg
    h = jnp.maximum(agg + s1_ref[...], 0.0)
    m_ref[...] = _bcos_block(h, wn_ref[...])
    s_ref[...] = lax.dot_general(h, ws_ref[...], (((1,), (1,)), ((), ())),
                                 preferred_element_type=jnp.float32)


def _tc3_body(p0_ref, p1_ref, d0_ref, d1_ref, s2_ref, o_ref):
    deg = jnp.maximum(d0_ref[:, 0:1] + d1_ref[:, 0:1], 1.0)
    o_ref[...] = (p0_ref[...] + p1_ref[...]) / deg + s2_ref[...]


def _row_block(n):
    for b in (2048, 2000, 1024, 1000, 512, 500, 400, 256, 200, 128, 80, 64,
              40, 32, 16, 8):
        if n % b == 0:
            return b
    return n


def _sc_mesh():
    return plsc.VectorSubcoreMesh(core_axis_name="c", subcore_axis_name="s")


def _make_sc_scatter(npad, cpw):
    assert cpw % 2 == 1
    rps = npad // NS
    nb = rps // K

    @functools.partial(
        pl.kernel, mesh=_sc_mesh(),
        out_type=[jax.ShapeDtypeStruct((NC, npad, 128), jnp.float32)],
        scratch_types=[
            pltpu.VMEM((K,), jnp.int32),
            pltpu.VMEM((K,), jnp.int32),
            pltpu.VMEM((K,), jnp.int32),
            pltpu.VMEM((K,), jnp.int32),
            pltpu.VMEM((K, 128), jnp.float32),
            pltpu.VMEM((K, 128), jnp.float32),
            pltpu.VMEM_SHARED((npad, 128), jnp.float32),
            pltpu.SemaphoreType.DMA,
            pltpu.SemaphoreType.DMA,
            pltpu.SemaphoreType.DMA,
            pltpu.SemaphoreType.DMA,
            pltpu.SemaphoreType.DMA,
            pltpu.SemaphoreType.DMA,
        ])
    def sc_fn(m_hbm, row_hbm, col_hbm, z_hbm, out_hbm,
              colv0, rowv0, colv1, rowv1, rows0, rows1, acc,
              sc0, sr0, sg0, sc1, sr1, sg1):
        cid = lax.axis_index("c")
        sid = lax.axis_index("s")
        wid = sid * NC + cid
        bufs = ((colv0, rowv0, rows0, sc0, sr0, sg0),
                (colv1, rowv1, rows1, sc1, sr1, sg1))

        pltpu.sync_copy(z_hbm, rows0)

        def zb(i, c):
            off = pl.multiple_of(sid * rps + i * K, K)
            pltpu.sync_copy(rows0, acc.at[pl.ds(off, K)])
            return c
        lax.fori_loop(0, nb, zb, 0)
        plsc.subcore_barrier()

        base = wid * cpw

        def start_idx(j, b):
            cv, rv, _, sc, sr, _ = bufs[b]
            off = pl.multiple_of((base + j) * K, K)
            pltpu.async_copy(col_hbm.at[pl.ds(off, K)], cv, sc)
            pltpu.async_copy(row_hbm.at[pl.ds(off, K)], rv, sr)

        def wait_idx(j, b):
            cv, rv, _, sc, sr, _ = bufs[b]
            off = pl.multiple_of((base + j) * K, K)
            pltpu.make_async_copy(col_hbm.at[pl.ds(off, K)], cv, sc).wait()
            pltpu.make_async_copy(row_hbm.at[pl.ds(off, K)], rv, sr).wait()

        def start_gather(b):
            cv, _, rw, _, _, sg = bufs[b]
            pltpu.async_copy(m_hbm.at[cv], rw, sg)

        def wait_gather(b):
            cv, _, rw, _, _, sg = bufs[b]
            pltpu.make_async_copy(m_hbm.at[cv], rw, sg).wait()

        def scatter(b):
            _, rv, rw, _, _, _ = bufs[b]
            pltpu.sync_copy(rw, acc.at[rv], add=True)

        def half(j, b, jpre):
            wait_gather(b)
            wait_idx(j + 1, 1 - b)
            start_gather(1 - b)
            scatter(b)
            start_idx(jpre, b)

        start_idx(0, 0)
        wait_idx(0, 0)
        start_idx(1, 1)
        start_gather(0)

        def pair(g, c):
            j = 2 * g
            half(j, 0, j + 2)
            half(j + 1, 1, jnp.minimum(j + 3, cpw - 1))
            return c
        lax.fori_loop(0, (cpw - 1) // 2, pair, 0)
        wait_gather(0)
        scatter(0)
        wait_idx(cpw - 1, 1)
        plsc.subcore_barrier()

        def wb(i, c):
            off = pl.multiple_of(sid * rps + i * K, K)
            pltpu.sync_copy(acc.at[pl.ds(off, K)], rows0)
            pltpu.sync_copy(rows0, out_hbm.at[cid, pl.ds(off, K)])
            return c
        lax.fori_loop(0, nb, wb, 0)

    return sc_fn


def _make_sc_deg(npad, cpw):
    rps = npad // NS
    nb = rps // K

    assert cpw % 2 == 1

    @functools.partial(
        pl.kernel, mesh=_sc_mesh(),
        out_type=[jax.ShapeDtypeStruct((NC, npad, 128), jnp.float32)],
        scratch_types=[
            pltpu.VMEM((K,), jnp.int32),
            pltpu.VMEM((K,), jnp.int32),
            pltpu.VMEM((K, 128), jnp.float32),
            pltpu.VMEM_SHARED((npad, 128), jnp.float32),
            pltpu.SemaphoreType.DMA,
            pltpu.SemaphoreType.DMA,
        ])
    def sc_fn(row_hbm, z_hbm, ones_hbm, out_hbm, rowv0, rowv1, rows, acc,
              sr0, sr1):
        cid = lax.axis_index("c")
        sid = lax.axis_index("s")
        wid = sid * NC + cid
        bufs = ((rowv0, sr0), (rowv1, sr1))

        pltpu.sync_copy(z_hbm, rows)

        def zb(i, c):
            off = pl.multiple_of(sid * rps + i * K, K)
            pltpu.sync_copy(rows, acc.at[pl.ds(off, K)])
            return c
        lax.fori_loop(0, nb, zb, 0)
        pltpu.sync_copy(ones_hbm, rows)
        plsc.subcore_barrier()

        base = wid * cpw

        def start_idx(j, b):
            rv, sr = bufs[b]
            off = pl.multiple_of((base + j) * K, K)
            pltpu.async_copy(row_hbm.at[pl.ds(off, K)], rv, sr)

        def wait_idx(j, b):
            rv, sr = bufs[b]
            off = pl.multiple_of((base + j) * K, K)
            pltpu.make_async_copy(row_hbm.at[pl.ds(off, K)], rv, sr).wait()

        def work(b):
            rv, _ = bufs[b]
            pltpu.sync_copy(rows, acc.at[rv], add=True)

        start_idx(0, 0)

        def pair(g, c):
            j = 2 * g
            wait_idx(j, 0)
            start_idx(j + 1, 1)
            work(0)
            wait_idx(j + 1, 1)
            start_idx(j + 2, 0)
            work(1)
            return c
        lax.fori_loop(0, (cpw - 1) // 2, pair, 0)
        wait_idx(cpw - 1, 0)
        work(0)
        plsc.subcore_barrier()

        def wb(i, c):
            off = pl.multiple_of(sid * rps + i * K, K)
            pltpu.sync_copy(acc.at[pl.ds(off, K)], rows)
            pltpu.sync_copy(rows, out_hbm.at[cid, pl.ds(off, K)])
            return c
        lax.fori_loop(0, nb, wb, 0)

    return sc_fn


def kernel(x, edge_index, Wn1, Wn2, Ws1, Ws2):
    n, d = x.shape
    e = edge_index.shape[1]
    nwork = NC * NS
    cpw = -(-e // (nwork * K))
    cpw += 1 - (cpw % 2)
    epad = nwork * cpw * K
    npad = -(-(n + 1) // (NS * K)) * NS * K

    row = edge_index[0]
    col = edge_index[1]
    pad = epad - e
    padrow = n + jnp.arange(pad, dtype=jnp.int32) % (npad - n)
    rowp = jnp.concatenate([row, padrow])
    colp = jnp.concatenate([col, jnp.zeros((pad,), jnp.int32)])
    z128 = jnp.zeros((K, 128), jnp.float32)
    o128 = jnp.ones((K, 128), jnp.float32)

    bm = _row_block(n)
    grid = (n // bm,)
    full_w = pl.BlockSpec((d, d), lambda i: (0, 0))
    rows128 = pl.BlockSpec((bm, 128), lambda i: (i, 0))

    (degp,) = _make_sc_deg(npad, cpw)(rowp, z128, o128)

    m1, s1 = pl.pallas_call(
        _tc1_body,
        grid=grid,
        in_specs=[rows128, full_w, full_w],
        out_specs=[rows128, rows128],
        out_shape=[jax.ShapeDtypeStruct((n, 128), jnp.float32)] * 2,
    )(x, Wn1, Ws1)

    sc_scatter = _make_sc_scatter(npad, cpw)
    (aggp,) = sc_scatter(m1, rowp, colp, z128)

    m2, s2 = pl.pallas_call(
        _tc2_body,
        grid=grid,
        in_specs=[rows128, rows128, rows128, rows128, rows128,
                  full_w, full_w],
        out_specs=[rows128, rows128],
        out_shape=[jax.ShapeDtypeStruct((n, 128), jnp.float32)] * 2,
    )(aggp[0, :n], aggp[1, :n], degp[0, :n], degp[1, :n], s1, Wn2, Ws2)

    (agg2p,) = sc_scatter(m2, rowp, colp, z128)

    out = pl.pallas_call(
        _tc3_body,
        grid=grid,
        in_specs=[rows128, rows128, rows128, rows128, rows128],
        out_specs=rows128,
        out_shape=jax.ShapeDtypeStruct((n, 128), jnp.float32),
    )(agg2p[0, :n], agg2p[1, :n], degp[0, :n], degp[1, :n], s2)

    return out
